# Initial kernel scaffold; baseline (speedup 1.0000x reference)
import jax
import jax.numpy as jnp
from jax.experimental import pallas as pl


def kernel(x, edge_index, batch, ptr, Wp, bp, Wl1, Wr1, att1, b1, g1, be1, Wl4, Wr4, att4, b4, g4, be4, Wlin, blin):
    raise NotImplementedError("write your pallas kernel here")


# speedup vs baseline: 32.3520x; 32.3520x over previous
import functools

import jax
import jax.numpy as jnp
from jax import lax
from jax.experimental import pallas as pl
from jax.experimental.pallas import tpu as pltpu
from jax.experimental.pallas import tpu_sc as plsc

N = 10000
TD = 128
HEADS = 8
HC = 16
NG = 16
NCLS = 4

NPAD = 10032
NCORES = 2
NSUB = 16
K = 128


def _sc_edge_pass(heads_local, ept, split_heads):
    d = heads_local * HC
    acc_w = 8 + d
    ch = ept // K
    rows_per_tile = NPAD // NSUB

    mesh = plsc.VectorSubcoreMesh(
        core_axis_name="c", subcore_axis_name="s",
        num_cores=NCORES, num_subcores=NSUB)

    @functools.partial(
        pl.kernel,
        out_type=jax.ShapeDtypeStruct((NCORES, NPAD, acc_w), jnp.float32),
        mesh=mesh,
        compiler_params=pltpu.CompilerParams(use_tc_tiling_on_sc=False),
        scratch_types=[
            pltpu.VMEM((1, K), jnp.int32),
            pltpu.VMEM((1, K), jnp.int32),
            pltpu.VMEM((1, K), jnp.int32),
            pltpu.VMEM((K, d), jnp.float32),
            pltpu.VMEM((K, d), jnp.float32),
            pltpu.VMEM((K, acc_w), jnp.float32),
            pltpu.VMEM((HEADS, HC), jnp.float32),
            pltpu.VMEM_SHARED((NPAD, acc_w), jnp.float32),
            pltpu.SemaphoreType.DMA,
            pltpu.SemaphoreType.DMA,
        ],
    )
    def kern(xl_hbm, xr_hbm, src_hbm, dst_hbm, att_hbm, out_hbm,
             src_v, dst_v, dstg_v, xj_v, xi_v, row_v, att_v, acc,
             sem1, sem2):
        c = lax.axis_index("c")
        s = lax.axis_index("s")

        pltpu.sync_copy(att_hbm, att_v)

        zoffs = list(range(0, acc_w - 15, 16))
        if acc_w % 16:
            zoffs.append(acc_w - 16)

        def zrow(i, carry):
            for j in zoffs:
                row_v[i, pl.ds(j, 16)] = jnp.zeros((16,), jnp.float32)
            return carry
        lax.fori_loop(0, K, zrow, 0)
        row0 = s * rows_per_tile
        nfull = rows_per_tile // K
        for t in range(nfull):
            pltpu.sync_copy(row_v, acc.at[pl.ds(row0 + t * K, K)])
        rem = rows_per_tile - nfull * K
        if rem:
            pltpu.sync_copy(row_v.at[pl.ds(0, rem)],
                            acc.at[pl.ds(row0 + nfull * K, rem)])
        plsc.subcore_barrier()

        lane = lax.iota(jnp.int32, 16)
        bfly = [jnp.bitwise_xor(lane, 1 << k) for k in range(4)]

        def allsum(vec):
            for idx in bfly:
                vec = vec + vec.at[idx].get(mode="promise_in_bounds")
            return vec

        if split_heads:
            edge_base = s * ept
        else:
            edge_base = (c * NSUB + s) * ept

        def chunk_body(ci, carry):
            off = edge_base + ci * K
            pltpu.sync_copy(src_hbm.at[pl.ds(off, K)], src_v.at[0])
            pltpu.sync_copy(dst_hbm.at[pl.ds(off, K)], dst_v.at[0])
            if split_heads:
                base = c * NPAD
                for j in range(K // 16):
                    sl = pl.ds(16 * j, 16)
                    src_v[0, sl] = src_v[0, sl] + base
                    dstg_v[0, sl] = dst_v[0, sl] + base
                gidx = dstg_v.at[0]
            else:
                gidx = dst_v.at[0]
            cp1 = pltpu.async_copy(xl_hbm.at[src_v.at[0]], xj_v, sem1)
            cp2 = pltpu.async_copy(xr_hbm.at[gidx], xi_v, sem2)
            cp1.wait()
            cp2.wait()

            def edge_body(e, ecarry):
                wden = jnp.zeros((16,), jnp.float32)
                numers = []
                for h in range(heads_local):
                    xj = xj_v[e, pl.ds(h * HC, HC)]
                    xi = xi_v[e, pl.ds(h * HC, HC)]
                    t = xi + xj
                    elr = jnp.where(t > 0, t, 0.2 * t)
                    if split_heads:
                        arow = att_v[c * heads_local + h, :]
                    else:
                        arow = att_v[h, :]
                    w = jnp.exp(allsum(elr * arow))
                    numers.append(w * xj)
                    wden = jnp.where(lane == h, w, wden)
                row_v[e, pl.ds(0, 16)] = wden
                for h in range(heads_local):
                    row_v[e, pl.ds(8 + h * HC, HC)] = numers[h]
                return ecarry
            lax.fori_loop(0, K, edge_body, 0)

            pltpu.sync_copy(row_v, acc.at[dst_v.at[0]], add=True)
            return carry
        lax.fori_loop(0, ch, chunk_body, 0)
        plsc.subcore_barrier()

        pltpu.sync_copy(acc.at[pl.ds(row0, rows_per_tile)],
                        out_hbm.at[c, pl.ds(row0, rows_per_tile)])

    return kern


def _tc_dense1(x, Wp, bp, Wl1, Wr1):
    blk = NPAD // 3
    dh = HEADS * HC // 2

    def body(x_ref, wp_ref, bp_ref, wl_ref, wr_ref, xl_ref, xr_ref):
        h = jnp.dot(x_ref[...], wp_ref[...],
                    preferred_element_type=jnp.float32) + bp_ref[...]
        xl = jnp.dot(h, wl_ref[...], preferred_element_type=jnp.float32)
        xr = jnp.dot(h, wr_ref[...], preferred_element_type=jnp.float32)
        xl_ref[0] = xl[:, :dh]
        xl_ref[1] = xl[:, dh:]
        xr_ref[0] = xr[:, :dh]
        xr_ref[1] = xr[:, dh:]

    return pl.pallas_call(
        body,
        grid=(NPAD // blk,),
        in_specs=[
            pl.BlockSpec((blk, TD), lambda i: (i, 0)),
            pl.BlockSpec((TD, HC), lambda i: (0, 0)),
            pl.BlockSpec((1, HC), lambda i: (0, 0)),
            pl.BlockSpec((HC, HEADS * HC), lambda i: (0, 0)),
            pl.BlockSpec((HC, HEADS * HC), lambda i: (0, 0)),
        ],
        out_specs=[
            pl.BlockSpec((2, blk, dh), lambda i: (0, i, 0)),
            pl.BlockSpec((2, blk, dh), lambda i: (0, i, 0)),
        ],
        out_shape=[jax.ShapeDtypeStruct((2, NPAD, dh), jnp.float32)] * 2,
    )(x, Wp, bp, Wl1, Wr1)


def _tc_dense2(acc, b1, g1, be1, Wl4, Wr4):
    blk = NPAD // 3
    d = HEADS * HC

    def body(a0_ref, a1_ref, b1_ref, g1_ref, be1_ref, wl_ref, wr_ref,
             xl_ref, xr_ref):
        halves = [a0_ref[0], a1_ref[0]]
        outs = []
        for h in range(HEADS):
            a = halves[h // 4]
            hl = h % 4
            num = a[:, 8 + HC * hl:8 + HC * (hl + 1)]
            den = a[:, hl:hl + 1]
            outs.append(num / (den + 1e-16))
        o = jnp.concatenate(outs, axis=1) + b1_ref[...]
        m = o.mean(-1, keepdims=True)
        v = ((o - m) ** 2).mean(-1, keepdims=True)
        h1 = (o - m) / jnp.sqrt(v + 1e-5) * g1_ref[...] + be1_ref[...]
        h1 = jnp.where(h1 > 0, h1, jnp.exp(h1) - 1.0)
        xl_ref[...] = jnp.dot(h1, wl_ref[...],
                              preferred_element_type=jnp.float32)
        xr_ref[...] = jnp.dot(h1, wr_ref[...],
                              preferred_element_type=jnp.float32)

    acc_w = 8 + 4 * HC
    return pl.pallas_call(
        body,
        grid=(NPAD // blk,),
        in_specs=[
            pl.BlockSpec((1, blk, acc_w), lambda i: (0, i, 0)),
            pl.BlockSpec((1, blk, acc_w), lambda i: (1, i, 0)),
            pl.BlockSpec((1, d), lambda i: (0, 0)),
            pl.BlockSpec((1, d), lambda i: (0, 0)),
            pl.BlockSpec((1, d), lambda i: (0, 0)),
            pl.BlockSpec((d, HC), lambda i: (0, 0)),
            pl.BlockSpec((d, HC), lambda i: (0, 0)),
        ],
        out_specs=[
            pl.BlockSpec((blk, HC), lambda i: (i, 0)),
            pl.BlockSpec((blk, HC), lambda i: (i, 0)),
        ],
        out_shape=[jax.ShapeDtypeStruct((NPAD, HC), jnp.float32)] * 2,
    )(acc, acc, b1, g1, be1, Wl4, Wr4)


def _tc_final(acc, ptr, b4, g4, be4, Wlin, blin):
    acc_w = 8 + HC

    def body(a0_ref, a1_ref, ptr_ref, b4_ref, g4_ref, be4_ref, wl_ref,
             bl_ref, out_ref, h4_scr, summ_scr):
        a = a0_ref[0] + a1_ref[0]
        num = a[:, 8:8 + HC]
        den = a[:, 0:1]
        o = num / (den + 1e-16) + b4_ref[...]
        m = o.mean(-1, keepdims=True)
        v = ((o - m) ** 2).mean(-1, keepdims=True)
        h4_scr[...] = (o - m) / jnp.sqrt(v + 1e-5) * g4_ref[...] + be4_ref[...]
        for j in range(NG):
            idx = ptr_ref[j]
            summ_scr[pl.ds(j, 1), :] = h4_scr[pl.ds(idx, 1), :]
        out_ref[...] = jnp.dot(summ_scr[...], wl_ref[...],
                               preferred_element_type=jnp.float32) + bl_ref[...]

    return pl.pallas_call(
        body,
        grid=(1,),
        in_specs=[
            pl.BlockSpec((1, NPAD, acc_w), lambda i: (0, 0, 0)),
            pl.BlockSpec((1, NPAD, acc_w), lambda i: (1, 0, 0)),
            pl.BlockSpec(memory_space=pltpu.SMEM),
            pl.BlockSpec((1, HC), lambda i: (0, 0)),
            pl.BlockSpec((1, HC), lambda i: (0, 0)),
            pl.BlockSpec((1, HC), lambda i: (0, 0)),
            pl.BlockSpec((HC, NCLS), lambda i: (0, 0)),
            pl.BlockSpec((1, NCLS), lambda i: (0, 0)),
        ],
        out_specs=pl.BlockSpec((NG, NCLS), lambda i: (0, 0)),
        out_shape=jax.ShapeDtypeStruct((NG, NCLS), jnp.float32),
        scratch_shapes=[
            pltpu.VMEM((NPAD, HC), jnp.float32),
            pltpu.VMEM((NG, HC), jnp.float32),
        ],
    )(acc, acc, ptr, b4, g4, be4, Wlin, blin)


def kernel(x, edge_index, batch, ptr, Wp, bp, Wl1, Wr1, att1, b1, g1, be1,
           Wl4, Wr4, att4, b4, g4, be4, Wlin, blin):
    e_tot = edge_index.shape[1] + N
    ept1 = -(-e_tot // (NSUB * K)) * K
    e_pad = ept1 * NSUB
    ept4 = e_pad // (NCORES * NSUB)

    loop = jnp.arange(N, dtype=jnp.int32)
    fill = jnp.full((e_pad - e_tot,), N, jnp.int32)
    src = jnp.concatenate([edge_index[0].astype(jnp.int32), loop, fill])
    dst = jnp.concatenate([edge_index[1].astype(jnp.int32), loop, fill])

    xpad = jnp.zeros((NPAD, TD), jnp.float32).at[:N].set(x)

    xls, xrs = _tc_dense1(xpad, Wp, bp.reshape(1, -1), Wl1, Wr1)
    xls = xls.reshape(2 * NPAD, HEADS * HC // 2)
    xrs = xrs.reshape(2 * NPAD, HEADS * HC // 2)
    att1h = att1.reshape(HEADS, HC)
    acc1 = _sc_edge_pass(4, ept1, True)(xls, xrs, src, dst, att1h)
    xl4, xr4 = _tc_dense2(acc1, b1.reshape(1, -1), g1.reshape(1, -1),
                          be1.reshape(1, -1), Wl4, Wr4)
    att4p = jnp.zeros((HEADS, HC), jnp.float32).at[0].set(att4[0])
    acc4 = _sc_edge_pass(1, ept4, False)(xl4, xr4, src, dst, att4p)
    return _tc_final(acc4, ptr, b4.reshape(1, -1), g4.reshape(1, -1),
                     be4.reshape(1, -1), Wlin, blin.reshape(1, -1))


# speedup vs baseline: 41.4793x; 1.2821x over previous
import functools

import jax
import jax.numpy as jnp
from jax import lax
from jax.experimental import pallas as pl
from jax.experimental.pallas import tpu as pltpu
from jax.experimental.pallas import tpu_sc as plsc

N = 10000
TD = 128
HEADS = 8
HC = 16
NG = 16
NCLS = 4

NPAD = 10032
NCORES = 2
NSUB = 16
K = 128


def _sc_edge_pass(heads_local, ept, split_heads):
    d = heads_local * HC
    acc_w = 8 + d
    ch = ept // K
    rows_per_tile = NPAD // NSUB
    unroll = 2 if split_heads else 4

    mesh = plsc.VectorSubcoreMesh(
        core_axis_name="c", subcore_axis_name="s",
        num_cores=NCORES, num_subcores=NSUB)

    @functools.partial(
        pl.kernel,
        out_type=jax.ShapeDtypeStruct((NCORES, NPAD, acc_w), jnp.float32),
        mesh=mesh,
        compiler_params=pltpu.CompilerParams(use_tc_tiling_on_sc=False),
        scratch_types=[
            pltpu.VMEM((2, K), jnp.int32),
            pltpu.VMEM((2, K), jnp.int32),
            pltpu.VMEM((2, K), jnp.int32),
            [pltpu.VMEM((K, d), jnp.float32)] * 2,
            [pltpu.VMEM((K, d), jnp.float32)] * 2,
            pltpu.VMEM((K, acc_w), jnp.float32),
            pltpu.VMEM((HEADS, HC), jnp.float32),
            pltpu.VMEM_SHARED((NPAD, acc_w), jnp.float32),
            [pltpu.SemaphoreType.DMA] * 2,
            [pltpu.SemaphoreType.DMA] * 2,
            [pltpu.SemaphoreType.DMA] * 2,
            [pltpu.SemaphoreType.DMA] * 2,
        ],
    )
    def kern(xl_hbm, xr_hbm, src_hbm, dst_hbm, att_hbm, out_hbm,
             sidx_v, didx_v, didxg_v, xj_v, xi_v, row_v, att_v, acc,
             semj, semi, semsi, semdi):
        c = lax.axis_index("c")
        s = lax.axis_index("s")

        pltpu.sync_copy(att_hbm, att_v)

        if split_heads:
            cbase = s * ch
        else:
            cbase = (c * NSUB + s) * ch

        zoffs = list(range(0, acc_w - 15, 16))
        if acc_w % 16:
            zoffs.append(acc_w - 16)

        def zrow(i, carry):
            for j in zoffs:
                row_v[i, pl.ds(j, 16)] = jnp.zeros((16,), jnp.float32)
            return carry
        lax.fori_loop(0, K, zrow, 0)
        row0 = s * rows_per_tile
        nfull = rows_per_tile // K
        for t in range(nfull):
            pltpu.sync_copy(row_v, acc.at[pl.ds(row0 + t * K, K)])
        rem = rows_per_tile - nfull * K
        if rem:
            pltpu.sync_copy(row_v.at[pl.ds(0, rem)],
                            acc.at[pl.ds(row0 + nfull * K, rem)])
        plsc.subcore_barrier()

        lane = lax.iota(jnp.int32, 16)
        bfly = [jnp.bitwise_xor(lane, 1 << k) for k in range(4)]

        def allsum(vec):
            for idx in bfly:
                vec = vec + vec.at[idx].get(mode="promise_in_bounds")
            return vec

        def idx_start(ci, b):
            pltpu.async_copy(src_hbm.at[cbase + ci], sidx_v.at[b], semsi[b])
            pltpu.async_copy(dst_hbm.at[cbase + ci], didx_v.at[b], semdi[b])

        def idx_wait(ci, b):
            pltpu.make_async_copy(
                src_hbm.at[cbase + ci], sidx_v.at[b], semsi[b]).wait()
            pltpu.make_async_copy(
                dst_hbm.at[cbase + ci], didx_v.at[b], semdi[b]).wait()
            if split_heads:
                base = c * NPAD
                for j in range(K // 16):
                    sl = pl.ds(16 * j, 16)
                    sidx_v[b, sl] = sidx_v[b, sl] + base
                    didxg_v[b, sl] = didx_v[b, sl] + base

        def row_start(ci, b):
            gd = didxg_v if split_heads else didx_v
            pltpu.async_copy(xl_hbm.at[sidx_v.at[b]], xj_v[b], semj[b])
            pltpu.async_copy(xr_hbm.at[gd.at[b]], xi_v[b], semi[b])

        def row_wait(ci, b):
            gd = didxg_v if split_heads else didx_v
            pltpu.make_async_copy(
                xl_hbm.at[sidx_v.at[b]], xj_v[b], semj[b]).wait()
            pltpu.make_async_copy(
                xr_hbm.at[gd.at[b]], xi_v[b], semi[b]).wait()

        def compute_scatter(ci, b):
            def edge_body(eu, ecarry):
                for u in range(unroll):
                    e = eu * unroll + u
                    wden = jnp.zeros((16,), jnp.float32)
                    numers = []
                    for h in range(heads_local):
                        xj = xj_v[b][e, pl.ds(h * HC, HC)]
                        xi = xi_v[b][e, pl.ds(h * HC, HC)]
                        t = xi + xj
                        elr = jnp.where(t > 0, t, 0.2 * t)
                        if split_heads:
                            arow = att_v[c * heads_local + h, :]
                        else:
                            arow = att_v[h, :]
                        w = jnp.exp(allsum(elr * arow))
                        numers.append(w * xj)
                        wden = jnp.where(lane == h, w, wden)
                    row_v[e, pl.ds(0, 16)] = wden
                    for h in range(heads_local):
                        row_v[e, pl.ds(8 + h * HC, HC)] = numers[h]
                return ecarry
            lax.fori_loop(0, K // unroll, edge_body, 0)
            pltpu.sync_copy(row_v, acc.at[didx_v.at[b]], add=True)

        idx_start(0, 0)
        idx_wait(0, 0)
        row_start(0, 0)
        if ch > 1:
            idx_start(1, 1)

        def pair_body(cg, carry):
            for b in range(2):
                ci = 2 * cg + b
                row_wait(ci, b)

                @pl.when(ci + 1 < ch)
                def _():
                    idx_wait(ci + 1, 1 - b)
                    row_start(ci + 1, 1 - b)
                compute_scatter(ci, b)

                @pl.when(ci + 2 < ch)
                def _():
                    idx_start(ci + 2, b)
            return carry
        lax.fori_loop(0, ch // 2, pair_body, 0)
        if ch % 2:
            ci = ch - 1
            row_wait(ci, ci % 2)
            compute_scatter(ci, ci % 2)
        plsc.subcore_barrier()

        pltpu.sync_copy(acc.at[pl.ds(row0, rows_per_tile)],
                        out_hbm.at[c, pl.ds(row0, rows_per_tile)])

    return kern


def _tc_dense1(x, Wp, bp, Wl1, Wr1):
    blk = NPAD // 3
    dh = HEADS * HC // 2

    def body(x_ref, wp_ref, bp_ref, wl_ref, wr_ref, xl_ref, xr_ref):
        h = jnp.dot(x_ref[...], wp_ref[...],
                    preferred_element_type=jnp.float32) + bp_ref[...]
        xl = jnp.dot(h, wl_ref[...], preferred_element_type=jnp.float32)
        xr = jnp.dot(h, wr_ref[...], preferred_element_type=jnp.float32)
        xl_ref[0] = xl[:, :dh]
        xl_ref[1] = xl[:, dh:]
        xr_ref[0] = xr[:, :dh]
        xr_ref[1] = xr[:, dh:]

    return pl.pallas_call(
        body,
        grid=(NPAD // blk,),
        in_specs=[
            pl.BlockSpec((blk, TD), lambda i: (i, 0)),
            pl.BlockSpec((TD, HC), lambda i: (0, 0)),
            pl.BlockSpec((1, HC), lambda i: (0, 0)),
            pl.BlockSpec((HC, HEADS * HC), lambda i: (0, 0)),
            pl.BlockSpec((HC, HEADS * HC), lambda i: (0, 0)),
        ],
        out_specs=[
            pl.BlockSpec((2, blk, dh), lambda i: (0, i, 0)),
            pl.BlockSpec((2, blk, dh), lambda i: (0, i, 0)),
        ],
        out_shape=[jax.ShapeDtypeStruct((2, NPAD, dh), jnp.float32)] * 2,
    )(x, Wp, bp, Wl1, Wr1)


def _tc_dense2(acc, b1, g1, be1, Wl4, Wr4):
    blk = NPAD // 3
    d = HEADS * HC

    def body(a0_ref, a1_ref, b1_ref, g1_ref, be1_ref, wl_ref, wr_ref,
             xl_ref, xr_ref):
        halves = [a0_ref[0], a1_ref[0]]
        outs = []
        for h in range(HEADS):
            a = halves[h // 4]
            hl = h % 4
            num = a[:, 8 + HC * hl:8 + HC * (hl + 1)]
            den = a[:, hl:hl + 1]
            outs.append(num / (den + 1e-16))
        o = jnp.concatenate(outs, axis=1) + b1_ref[...]
        m = o.mean(-1, keepdims=True)
        v = ((o - m) ** 2).mean(-1, keepdims=True)
        h1 = (o - m) / jnp.sqrt(v + 1e-5) * g1_ref[...] + be1_ref[...]
        h1 = jnp.where(h1 > 0, h1, jnp.exp(h1) - 1.0)
        xl_ref[...] = jnp.dot(h1, wl_ref[...],
                              preferred_element_type=jnp.float32)
        xr_ref[...] = jnp.dot(h1, wr_ref[...],
                              preferred_element_type=jnp.float32)

    acc_w = 8 + 4 * HC
    return pl.pallas_call(
        body,
        grid=(NPAD // blk,),
        in_specs=[
            pl.BlockSpec((1, blk, acc_w), lambda i: (0, i, 0)),
            pl.BlockSpec((1, blk, acc_w), lambda i: (1, i, 0)),
            pl.BlockSpec((1, d), lambda i: (0, 0)),
            pl.BlockSpec((1, d), lambda i: (0, 0)),
            pl.BlockSpec((1, d), lambda i: (0, 0)),
            pl.BlockSpec((d, HC), lambda i: (0, 0)),
            pl.BlockSpec((d, HC), lambda i: (0, 0)),
        ],
        out_specs=[
            pl.BlockSpec((blk, HC), lambda i: (i, 0)),
            pl.BlockSpec((blk, HC), lambda i: (i, 0)),
        ],
        out_shape=[jax.ShapeDtypeStruct((NPAD, HC), jnp.float32)] * 2,
    )(acc, acc, b1, g1, be1, Wl4, Wr4)


def _tc_final(acc, ptr, b4, g4, be4, Wlin, blin):
    acc_w = 8 + HC

    def body(a0_ref, a1_ref, ptr_ref, b4_ref, g4_ref, be4_ref, wl_ref,
             bl_ref, out_ref, h4_scr, summ_scr):
        a = a0_ref[0] + a1_ref[0]
        num = a[:, 8:8 + HC]
        den = a[:, 0:1]
        o = num / (den + 1e-16) + b4_ref[...]
        m = o.mean(-1, keepdims=True)
        v = ((o - m) ** 2).mean(-1, keepdims=True)
        h4_scr[...] = (o - m) / jnp.sqrt(v + 1e-5) * g4_ref[...] + be4_ref[...]
        for j in range(NG):
            idx = ptr_ref[j]
            summ_scr[pl.ds(j, 1), :] = h4_scr[pl.ds(idx, 1), :]
        out_ref[...] = jnp.dot(summ_scr[...], wl_ref[...],
                               preferred_element_type=jnp.float32) + bl_ref[...]

    return pl.pallas_call(
        body,
        grid=(1,),
        in_specs=[
            pl.BlockSpec((1, NPAD, acc_w), lambda i: (0, 0, 0)),
            pl.BlockSpec((1, NPAD, acc_w), lambda i: (1, 0, 0)),
            pl.BlockSpec(memory_space=pltpu.SMEM),
            pl.BlockSpec((1, HC), lambda i: (0, 0)),
            pl.BlockSpec((1, HC), lambda i: (0, 0)),
            pl.BlockSpec((1, HC), lambda i: (0, 0)),
            pl.BlockSpec((HC, NCLS), lambda i: (0, 0)),
            pl.BlockSpec((1, NCLS), lambda i: (0, 0)),
        ],
        out_specs=pl.BlockSpec((NG, NCLS), lambda i: (0, 0)),
        out_shape=jax.ShapeDtypeStruct((NG, NCLS), jnp.float32),
        scratch_shapes=[
            pltpu.VMEM((NPAD, HC), jnp.float32),
            pltpu.VMEM((NG, HC), jnp.float32),
        ],
    )(acc, acc, ptr, b4, g4, be4, Wlin, blin)


def kernel(x, edge_index, batch, ptr, Wp, bp, Wl1, Wr1, att1, b1, g1, be1,
           Wl4, Wr4, att4, b4, g4, be4, Wlin, blin):
    e_tot = edge_index.shape[1] + N
    ept1 = -(-e_tot // (NSUB * K)) * K
    e_pad = ept1 * NSUB
    ept4 = e_pad // (NCORES * NSUB)

    loop = jnp.arange(N, dtype=jnp.int32)
    fill = jnp.full((e_pad - e_tot,), N, jnp.int32)
    src = jnp.concatenate(
        [edge_index[0].astype(jnp.int32), loop, fill]).reshape(-1, K)
    dst = jnp.concatenate(
        [edge_index[1].astype(jnp.int32), loop, fill]).reshape(-1, K)

    xpad = jnp.zeros((NPAD, TD), jnp.float32).at[:N].set(x)

    xls, xrs = _tc_dense1(xpad, Wp, bp.reshape(1, -1), Wl1, Wr1)
    xls = xls.reshape(2 * NPAD, HEADS * HC // 2)
    xrs = xrs.reshape(2 * NPAD, HEADS * HC // 2)
    att1h = att1.reshape(HEADS, HC)
    acc1 = _sc_edge_pass(4, ept1, True)(xls, xrs, src, dst, att1h)
    xl4, xr4 = _tc_dense2(acc1, b1.reshape(1, -1), g1.reshape(1, -1),
                          be1.reshape(1, -1), Wl4, Wr4)
    att4p = jnp.zeros((HEADS, HC), jnp.float32).at[0].set(att4[0])
    acc4 = _sc_edge_pass(1, ept4, False)(xl4, xr4, src, dst, att4p)
    return _tc_final(acc4, ptr, b4.reshape(1, -1), g4.reshape(1, -1),
                     be4.reshape(1, -1), Wlin, blin.reshape(1, -1))


# speedup vs baseline: 71.5630x; 1.7253x over previous
import functools

import jax
import jax.numpy as jnp
from jax import lax
from jax.experimental import pallas as pl
from jax.experimental.pallas import tpu as pltpu
from jax.experimental.pallas import tpu_sc as plsc

N = 10000
TD = 128
HEADS = 8
HC = 16
NG = 16
NCLS = 4

NPAD = 10032
NCORES = 2
NSUB = 16
K = 128


def _sc_edge_pass(heads_local, ept, split_heads):
    d = heads_local * HC
    acc_w = 8 + d
    ch = ept // K
    rows_per_tile = NPAD // NSUB
    unroll = 4 if split_heads else 8

    mesh = plsc.VectorSubcoreMesh(
        core_axis_name="c", subcore_axis_name="s",
        num_cores=NCORES, num_subcores=NSUB)

    @functools.partial(
        pl.kernel,
        out_type=jax.ShapeDtypeStruct((NCORES, NPAD, acc_w), jnp.float32),
        mesh=mesh,
        compiler_params=pltpu.CompilerParams(use_tc_tiling_on_sc=False),
        scratch_types=[
            pltpu.VMEM((2, K), jnp.int32),
            pltpu.VMEM((2, K), jnp.int32),
            pltpu.VMEM((2, K), jnp.int32),
            [pltpu.VMEM((K, d), jnp.float32)] * 2,
            [pltpu.VMEM((K, d), jnp.float32)] * 2,
            pltpu.VMEM((K, acc_w), jnp.float32),
            pltpu.VMEM((HEADS, HC), jnp.float32),
            pltpu.VMEM_SHARED((NPAD, acc_w), jnp.float32),
            [pltpu.SemaphoreType.DMA] * 2,
            [pltpu.SemaphoreType.DMA] * 2,
            [pltpu.SemaphoreType.DMA] * 2,
            [pltpu.SemaphoreType.DMA] * 2,
        ],
    )
    def kern(xl_hbm, xr_hbm, src_hbm, dst_hbm, att_hbm, out_hbm,
             sidx_v, didx_v, didxg_v, xj_v, xi_v, row_v, att_v, acc,
             semj, semi, semsi, semdi):
        c = lax.axis_index("c")
        s = lax.axis_index("s")

        pltpu.sync_copy(att_hbm, att_v)

        if split_heads:
            cbase = s * ch
        else:
            cbase = (c * NSUB + s) * ch

        zoffs = list(range(0, acc_w - 15, 16))
        if acc_w % 16:
            zoffs.append(acc_w - 16)

        def zrow(i, carry):
            for j in zoffs:
                row_v[i, pl.ds(j, 16)] = jnp.zeros((16,), jnp.float32)
            return carry
        lax.fori_loop(0, K, zrow, 0)
        row0 = s * rows_per_tile
        nfull = rows_per_tile // K
        for t in range(nfull):
            pltpu.sync_copy(row_v, acc.at[pl.ds(row0 + t * K, K)])
        rem = rows_per_tile - nfull * K
        if rem:
            pltpu.sync_copy(row_v.at[pl.ds(0, rem)],
                            acc.at[pl.ds(row0 + nfull * K, rem)])
        plsc.subcore_barrier()

        lane = lax.iota(jnp.int32, 16)
        bfly = [jnp.bitwise_xor(lane, 1 << k) for k in range(4)]

        def allsum(vec):
            for idx in bfly:
                vec = vec + vec.at[idx].get(mode="promise_in_bounds")
            return vec

        def idx_start(ci, b):
            pltpu.async_copy(src_hbm.at[cbase + ci], sidx_v.at[b], semsi[b])
            pltpu.async_copy(dst_hbm.at[cbase + ci], didx_v.at[b], semdi[b])

        def idx_wait(ci, b):
            pltpu.make_async_copy(
                src_hbm.at[cbase + ci], sidx_v.at[b], semsi[b]).wait()
            pltpu.make_async_copy(
                dst_hbm.at[cbase + ci], didx_v.at[b], semdi[b]).wait()
            if split_heads:
                base = c * NPAD
                for j in range(K // 16):
                    sl = pl.ds(16 * j, 16)
                    sidx_v[b, sl] = sidx_v[b, sl] + base
                    didxg_v[b, sl] = didx_v[b, sl] + base

        def row_start(ci, b):
            gd = didxg_v if split_heads else didx_v
            pltpu.async_copy(xl_hbm.at[sidx_v.at[b]], xj_v[b], semj[b])
            pltpu.async_copy(xr_hbm.at[gd.at[b]], xi_v[b], semi[b])

        def row_wait(ci, b):
            gd = didxg_v if split_heads else didx_v
            pltpu.make_async_copy(
                xl_hbm.at[sidx_v.at[b]], xj_v[b], semj[b]).wait()
            pltpu.make_async_copy(
                xr_hbm.at[gd.at[b]], xi_v[b], semi[b]).wait()

        if split_heads:
            arows = [att_v[c * heads_local + h, :]
                     for h in range(heads_local)]
        else:
            arows = [att_v[h, :] for h in range(heads_local)]
        a06 = [0.6 * a for a in arows]
        a04 = [0.4 * a for a in arows]
        hmasks = [lane == 8 + h for h in range(heads_local)]
        l8 = lane < 8
        swap8 = bfly[3]

        def shift8(x):
            return x.at[swap8].get(mode="promise_in_bounds")

        def compute_scatter(ci, b):
            @plsc.parallel_loop(0, K, step=1, unroll=unroll)
            def edge_body(e):
                wden = jnp.zeros((16,), jnp.float32)
                numers = []
                for h in range(heads_local):
                    xj = xj_v[b][e, pl.ds(h * HC, HC)]
                    xi = xi_v[b][e, pl.ds(h * HC, HC)]
                    t = xi + xj
                    prod = t * a06[h] + jnp.abs(t) * a04[h]
                    w = jnp.exp(allsum(prod))
                    numers.append(w * xj)
                    wden = jnp.where(hmasks[h], w, wden)
                parts = [wden] + numers
                for h in range(heads_local):
                    merged = shift8(jnp.where(l8, parts[h + 1], parts[h]))
                    row_v[e, pl.ds(16 * h, 16)] = merged
                row_v[e, pl.ds(8 + 16 * (heads_local - 1), 16)] = numers[-1]
            pltpu.sync_copy(row_v, acc.at[didx_v.at[b]], add=True)

        idx_start(0, 0)
        idx_wait(0, 0)
        row_start(0, 0)
        if ch > 1:
            idx_start(1, 1)

        def pair_body(cg, carry):
            for b in range(2):
                ci = 2 * cg + b
                row_wait(ci, b)

                @pl.when(ci + 1 < ch)
                def _():
                    idx_wait(ci + 1, 1 - b)
                    row_start(ci + 1, 1 - b)
                compute_scatter(ci, b)

                @pl.when(ci + 2 < ch)
                def _():
                    idx_start(ci + 2, b)
            return carry
        lax.fori_loop(0, ch // 2, pair_body, 0)
        if ch % 2:
            ci = ch - 1
            row_wait(ci, ci % 2)
            compute_scatter(ci, ci % 2)
        plsc.subcore_barrier()

        pltpu.sync_copy(acc.at[pl.ds(row0, rows_per_tile)],
                        out_hbm.at[c, pl.ds(row0, rows_per_tile)])

    return kern


def _tc_dense1(x, Wp, bp, Wl1, Wr1):
    blk = NPAD // 3
    dh = HEADS * HC // 2

    def body(x_ref, wp_ref, bp_ref, wl_ref, wr_ref, xl_ref, xr_ref):
        h = jnp.dot(x_ref[...], wp_ref[...],
                    preferred_element_type=jnp.float32) + bp_ref[...]
        xl = jnp.dot(h, wl_ref[...], preferred_element_type=jnp.float32)
        xr = jnp.dot(h, wr_ref[...], preferred_element_type=jnp.float32)
        xl_ref[0] = xl[:, :dh]
        xl_ref[1] = xl[:, dh:]
        xr_ref[0] = xr[:, :dh]
        xr_ref[1] = xr[:, dh:]

    return pl.pallas_call(
        body,
        grid=(NPAD // blk,),
        in_specs=[
            pl.BlockSpec((blk, TD), lambda i: (i, 0)),
            pl.BlockSpec((TD, HC), lambda i: (0, 0)),
            pl.BlockSpec((1, HC), lambda i: (0, 0)),
            pl.BlockSpec((HC, HEADS * HC), lambda i: (0, 0)),
            pl.BlockSpec((HC, HEADS * HC), lambda i: (0, 0)),
        ],
        out_specs=[
            pl.BlockSpec((2, blk, dh), lambda i: (0, i, 0)),
            pl.BlockSpec((2, blk, dh), lambda i: (0, i, 0)),
        ],
        out_shape=[jax.ShapeDtypeStruct((2, NPAD, dh), jnp.float32)] * 2,
    )(x, Wp, bp, Wl1, Wr1)


def _tc_dense2(acc, b1, g1, be1, Wl4, Wr4):
    blk = NPAD // 3
    d = HEADS * HC

    def body(a0_ref, a1_ref, b1_ref, g1_ref, be1_ref, wl_ref, wr_ref,
             xl_ref, xr_ref):
        halves = [a0_ref[0], a1_ref[0]]
        outs = []
        for h in range(HEADS):
            a = halves[h // 4]
            hl = h % 4
            num = a[:, 8 + HC * hl:8 + HC * (hl + 1)]
            den = a[:, hl:hl + 1]
            outs.append(num / (den + 1e-16))
        o = jnp.concatenate(outs, axis=1) + b1_ref[...]
        m = o.mean(-1, keepdims=True)
        v = ((o - m) ** 2).mean(-1, keepdims=True)
        h1 = (o - m) / jnp.sqrt(v + 1e-5) * g1_ref[...] + be1_ref[...]
        h1 = jnp.where(h1 > 0, h1, jnp.exp(h1) - 1.0)
        xl_ref[...] = jnp.dot(h1, wl_ref[...],
                              preferred_element_type=jnp.float32)
        xr_ref[...] = jnp.dot(h1, wr_ref[...],
                              preferred_element_type=jnp.float32)

    acc_w = 8 + 4 * HC
    return pl.pallas_call(
        body,
        grid=(NPAD // blk,),
        in_specs=[
            pl.BlockSpec((1, blk, acc_w), lambda i: (0, i, 0)),
            pl.BlockSpec((1, blk, acc_w), lambda i: (1, i, 0)),
            pl.BlockSpec((1, d), lambda i: (0, 0)),
            pl.BlockSpec((1, d), lambda i: (0, 0)),
            pl.BlockSpec((1, d), lambda i: (0, 0)),
            pl.BlockSpec((d, HC), lambda i: (0, 0)),
            pl.BlockSpec((d, HC), lambda i: (0, 0)),
        ],
        out_specs=[
            pl.BlockSpec((blk, HC), lambda i: (i, 0)),
            pl.BlockSpec((blk, HC), lambda i: (i, 0)),
        ],
        out_shape=[jax.ShapeDtypeStruct((NPAD, HC), jnp.float32)] * 2,
    )(acc, acc, b1, g1, be1, Wl4, Wr4)


def _tc_final(acc, ptr, b4, g4, be4, Wlin, blin):
    acc_w = 8 + HC

    def body(a0_ref, a1_ref, ptr_ref, b4_ref, g4_ref, be4_ref, wl_ref,
             bl_ref, out_ref, h4_scr, summ_scr):
        a = a0_ref[0] + a1_ref[0]
        num = a[:, 8:8 + HC]
        den = a[:, 0:1]
        o = num / (den + 1e-16) + b4_ref[...]
        m = o.mean(-1, keepdims=True)
        v = ((o - m) ** 2).mean(-1, keepdims=True)
        h4_scr[...] = (o - m) / jnp.sqrt(v + 1e-5) * g4_ref[...] + be4_ref[...]
        for j in range(NG):
            idx = ptr_ref[j]
            summ_scr[pl.ds(j, 1), :] = h4_scr[pl.ds(idx, 1), :]
        out_ref[...] = jnp.dot(summ_scr[...], wl_ref[...],
                               preferred_element_type=jnp.float32) + bl_ref[...]

    return pl.pallas_call(
        body,
        grid=(1,),
        in_specs=[
            pl.BlockSpec((1, NPAD, acc_w), lambda i: (0, 0, 0)),
            pl.BlockSpec((1, NPAD, acc_w), lambda i: (1, 0, 0)),
            pl.BlockSpec(memory_space=pltpu.SMEM),
            pl.BlockSpec((1, HC), lambda i: (0, 0)),
            pl.BlockSpec((1, HC), lambda i: (0, 0)),
            pl.BlockSpec((1, HC), lambda i: (0, 0)),
            pl.BlockSpec((HC, NCLS), lambda i: (0, 0)),
            pl.BlockSpec((1, NCLS), lambda i: (0, 0)),
        ],
        out_specs=pl.BlockSpec((NG, NCLS), lambda i: (0, 0)),
        out_shape=jax.ShapeDtypeStruct((NG, NCLS), jnp.float32),
        scratch_shapes=[
            pltpu.VMEM((NPAD, HC), jnp.float32),
            pltpu.VMEM((NG, HC), jnp.float32),
        ],
    )(acc, acc, ptr, b4, g4, be4, Wlin, blin)


def kernel(x, edge_index, batch, ptr, Wp, bp, Wl1, Wr1, att1, b1, g1, be1,
           Wl4, Wr4, att4, b4, g4, be4, Wlin, blin):
    e_tot = edge_index.shape[1] + N
    ept1 = -(-e_tot // (NSUB * K)) * K
    e_pad = ept1 * NSUB
    ept4 = e_pad // (NCORES * NSUB)

    loop = jnp.arange(N, dtype=jnp.int32)
    fill = jnp.full((e_pad - e_tot,), N, jnp.int32)
    src = jnp.concatenate(
        [edge_index[0].astype(jnp.int32), loop, fill]).reshape(-1, K)
    dst = jnp.concatenate(
        [edge_index[1].astype(jnp.int32), loop, fill]).reshape(-1, K)

    xpad = jnp.zeros((NPAD, TD), jnp.float32).at[:N].set(x)

    xls, xrs = _tc_dense1(xpad, Wp, bp.reshape(1, -1), Wl1, Wr1)
    xls = xls.reshape(2 * NPAD, HEADS * HC // 2)
    xrs = xrs.reshape(2 * NPAD, HEADS * HC // 2)
    att1h = att1.reshape(HEADS, HC)
    acc1 = _sc_edge_pass(4, ept1, True)(xls, xrs, src, dst, att1h)
    xl4, xr4 = _tc_dense2(acc1, b1.reshape(1, -1), g1.reshape(1, -1),
                          be1.reshape(1, -1), Wl4, Wr4)
    att4p = jnp.zeros((HEADS, HC), jnp.float32).at[0].set(att4[0])
    acc4 = _sc_edge_pass(1, ept4, False)(xl4, xr4, src, dst, att4p)
    return _tc_final(acc4, ptr, b4.reshape(1, -1), g4.reshape(1, -1),
                     be4.reshape(1, -1), Wlin, blin.reshape(1, -1))
